# baseline (speedup 1.0000x reference)
import functools

import jax
import jax.numpy as jnp
from jax.experimental import pallas as pl

N_T = 8192
N_S = 8192
DIM = 128

BM = 512


def _fc_kernel(source_ref, wt_ref, b_ref, out_ref):
    acc = jnp.dot(source_ref[...], wt_ref[...],
                  preferred_element_type=jnp.float32)
    out_ref[...] = jnp.maximum(acc + b_ref[...], 0.0)


def _collect_kernel(att_ref, fc_ref, out_ref):
    a = att_ref[...]
    acc = jnp.dot(a, fc_ref[...], preferred_element_type=jnp.float32)
    denom = jnp.sum(a, axis=1, keepdims=True) + 1e-7
    out_ref[...] = acc / denom


@jax.jit
def _run(source, attention, W, b):
    wt = W.T
    b2 = b.reshape(1, DIM)
    fc_out = pl.pallas_call(
        _fc_kernel,
        out_shape=jax.ShapeDtypeStruct((N_S, DIM), jnp.float32),
    )(source, wt, b2)

    out = pl.pallas_call(
        _collect_kernel,
        grid=(N_T // BM,),
        in_specs=[
            pl.BlockSpec((BM, N_S), lambda i: (i, 0)),
            pl.BlockSpec((N_S, DIM), lambda i: (0, 0)),
        ],
        out_specs=pl.BlockSpec((BM, DIM), lambda i: (i, 0)),
        out_shape=jax.ShapeDtypeStruct((N_T, DIM), jnp.float32),
    )(attention, fc_out)
    return out


def kernel(target, source, attention, W, b, unit_id):
    return _run(source, attention, W, b)


# speedup vs baseline: 1.0791x; 1.0791x over previous
import jax
import jax.numpy as jnp
from jax.experimental import pallas as pl
from jax.experimental.pallas import tpu as pltpu

N_T = 8192
N_S = 8192
DIM = 128

BM = 256


def _fused_kernel(att_ref, source_ref, wt_ref, b_ref, out_ref, fc_ref):
    @pl.when(pl.program_id(0) == 0)
    def _():
        acc = jnp.dot(source_ref[...], wt_ref[...],
                      preferred_element_type=jnp.float32)
        fc_ref[...] = jnp.maximum(acc + b_ref[...], 0.0)

    a = att_ref[...]
    acc = jnp.dot(a, fc_ref[...], preferred_element_type=jnp.float32)
    denom = jnp.sum(a, axis=1, keepdims=True) + 1e-7
    out_ref[...] = acc / denom


@jax.jit
def _run(source, attention, W, b):
    wt = W.T
    b2 = b.reshape(1, DIM)
    out = pl.pallas_call(
        _fused_kernel,
        grid=(N_T // BM,),
        in_specs=[
            pl.BlockSpec((BM, N_S), lambda i: (i, 0)),
            pl.BlockSpec((N_S, DIM), lambda i: (0, 0)),
            pl.BlockSpec((DIM, DIM), lambda i: (0, 0)),
            pl.BlockSpec((1, DIM), lambda i: (0, 0)),
        ],
        out_specs=pl.BlockSpec((BM, DIM), lambda i: (i, 0)),
        out_shape=jax.ShapeDtypeStruct((N_T, DIM), jnp.float32),
        scratch_shapes=[pltpu.VMEM((N_S, DIM), jnp.float32)],
    )(attention, source, wt, b2)
    return out


def kernel(target, source, attention, W, b, unit_id):
    return _run(source, attention, W, b)
